# baseline (speedup 1.0000x reference)
import jax
import jax.numpy as jnp
from jax.experimental import pallas as pl
from jax.experimental.pallas import tpu as pltpu

_C = 256
_NBUF = 8


def _load(x_hbm, x_bufs, load_sems, j, s):
    return pltpu.make_async_copy(
        x_hbm.at[pl.ds(j * _C, _C), :],
        x_bufs.at[pl.ds(s * _C, _C), :],
        load_sems.at[s],
    )


def _store(o_bufs, out_hbm, store_sems, j, s):
    return pltpu.make_async_copy(
        o_bufs.at[pl.ds(s * _C, _C), :],
        out_hbm.at[pl.ds(j * _C, _C), :],
        store_sems.at[s],
    )


def _body(idx_ref, w_ref, x_hbm, out_hbm, x_bufs, o_bufs, load_sems,
          store_sems):
    n_rows = x_hbm.shape[0]
    n_chunks = n_rows // _C

    for j in range(_NBUF):
        _load(x_hbm, x_bufs, load_sems, j, j).start()

    def step(i, carry):
        s = jax.lax.rem(i, _NBUF)
        _load(x_hbm, x_bufs, load_sems, i, s).wait()

        @pl.when(i >= _NBUF)
        def _():
            _store(o_bufs, out_hbm, store_sems, i - _NBUF, s).wait()

        scale = idx_ref[pl.ds(i, 1), :][0, :].astype(jnp.float32)[:, None]
        o_bufs[pl.ds(s * _C, _C), :] = (
            x_bufs[pl.ds(s * _C, _C), :] + scale * w_ref[...])
        _store(o_bufs, out_hbm, store_sems, i, s).start()

        @pl.when(i + _NBUF < n_chunks)
        def _():
            _load(x_hbm, x_bufs, load_sems, i + _NBUF, s).start()

        return carry

    jax.lax.fori_loop(0, n_chunks, step, 0)

    for j in range(n_chunks - _NBUF, n_chunks):
        _store(o_bufs, out_hbm, store_sems, j, j % _NBUF).wait()


def kernel(x, indices, embs_weight):
    B, S, D = x.shape
    n_rows = B * S
    n_chunks = n_rows // _C
    x2 = x.reshape(n_rows, D)
    idx2 = indices.reshape(n_chunks, _C)
    out = pl.pallas_call(
        _body,
        in_specs=[
            pl.BlockSpec(memory_space=pltpu.VMEM),
            pl.BlockSpec(memory_space=pltpu.VMEM),
            pl.BlockSpec(memory_space=pl.ANY),
        ],
        out_specs=pl.BlockSpec(memory_space=pl.ANY),
        out_shape=jax.ShapeDtypeStruct((n_rows, D), x.dtype),
        scratch_shapes=[
            pltpu.VMEM((_NBUF * _C, D), jnp.float32),
            pltpu.VMEM((_NBUF * _C, D), jnp.float32),
            pltpu.SemaphoreType.DMA((_NBUF,)),
            pltpu.SemaphoreType.DMA((_NBUF,)),
        ],
    )(idx2, embs_weight, x2)
    return out.reshape(B, S, D)


# speedup vs baseline: 1.0067x; 1.0067x over previous
import jax
import jax.numpy as jnp
from jax.experimental import pallas as pl
from jax.experimental.pallas import tpu as pltpu

_C = 1024
_NBUF = 4


def _load(x_hbm, x_bufs, load_sems, j, s):
    return pltpu.make_async_copy(
        x_hbm.at[pl.ds(j * _C, _C), :],
        x_bufs.at[pl.ds(s * _C, _C), :],
        load_sems.at[s],
    )


def _store(o_bufs, out_hbm, store_sems, j, s):
    return pltpu.make_async_copy(
        o_bufs.at[pl.ds(s * _C, _C), :],
        out_hbm.at[pl.ds(j * _C, _C), :],
        store_sems.at[s],
    )


def _body(idx_ref, w_ref, x_hbm, out_hbm, x_bufs, o_bufs, load_sems,
          store_sems):
    n_rows = x_hbm.shape[0]
    n_chunks = n_rows // _C

    for j in range(_NBUF):
        _load(x_hbm, x_bufs, load_sems, j, j).start()

    def step(i, carry):
        s = jax.lax.rem(i, _NBUF)
        _load(x_hbm, x_bufs, load_sems, i, s).wait()

        @pl.when(i >= _NBUF)
        def _():
            _store(o_bufs, out_hbm, store_sems, i - _NBUF, s).wait()

        scale = idx_ref[pl.ds(i, 1), :][0, :].astype(jnp.float32)[:, None]
        o_bufs[pl.ds(s * _C, _C), :] = (
            x_bufs[pl.ds(s * _C, _C), :] + scale * w_ref[...])
        _store(o_bufs, out_hbm, store_sems, i, s).start()

        @pl.when(i + _NBUF < n_chunks)
        def _():
            _load(x_hbm, x_bufs, load_sems, i + _NBUF, s).start()

        return carry

    jax.lax.fori_loop(0, n_chunks, step, 0)

    for j in range(n_chunks - _NBUF, n_chunks):
        _store(o_bufs, out_hbm, store_sems, j, j % _NBUF).wait()


def kernel(x, indices, embs_weight):
    B, S, D = x.shape
    n_rows = B * S
    n_chunks = n_rows // _C
    x2 = x.reshape(n_rows, D)
    idx2 = indices.reshape(n_chunks, _C)
    out = pl.pallas_call(
        _body,
        in_specs=[
            pl.BlockSpec(memory_space=pltpu.VMEM),
            pl.BlockSpec(memory_space=pltpu.VMEM),
            pl.BlockSpec(memory_space=pl.ANY),
        ],
        out_specs=pl.BlockSpec(memory_space=pl.ANY),
        out_shape=jax.ShapeDtypeStruct((n_rows, D), x.dtype),
        scratch_shapes=[
            pltpu.VMEM((_NBUF * _C, D), jnp.float32),
            pltpu.VMEM((_NBUF * _C, D), jnp.float32),
            pltpu.SemaphoreType.DMA((_NBUF,)),
            pltpu.SemaphoreType.DMA((_NBUF,)),
        ],
    )(idx2, embs_weight, x2)
    return out.reshape(B, S, D)
